# baseline (speedup 1.0000x reference)
import functools

import jax
import jax.numpy as jnp
import numpy as np
from jax import lax
from jax.experimental import pallas as pl
from jax.experimental.pallas import tpu as pltpu
from jax.experimental.pallas import tpu_sc as plsc

INPUT_DIM = 100000
OUTPUT_DIM = 64
BATCH = 4096
SEQ = 200
HALF = SEQ // 2
SCALE = float(np.sqrt(np.float32(OUTPUT_DIM)))


def _pos_encoding(position, d_model):
    i = np.arange(d_model)[np.newaxis, :]
    angle_rates = 1 / np.power(10000, 2 * (i // 2) / np.float32(d_model))
    angle_rads = np.arange(position)[:, np.newaxis] * angle_rates
    angle_rads[:, 0::2] = np.sin(angle_rads[:, 0::2])
    angle_rads[:, 1::2] = np.cos(angle_rads[:, 1::2])
    return np.asarray(angle_rads, dtype=np.float32)


_POS = _pos_encoding(SEQ, OUTPUT_DIM)


@functools.cache
def _build_kernel(nc, ns):
    nw = nc * ns
    total_rows = BATCH * SEQ
    rows_per_w = total_rows // nw
    seqs_per_w = rows_per_w // SEQ
    chunks_per_w = rows_per_w // HALF

    mesh = plsc.VectorSubcoreMesh(
        core_axis_name="c", subcore_axis_name="s",
        num_cores=nc, num_subcores=ns)

    @functools.partial(
        pl.kernel,
        out_type=jax.ShapeDtypeStruct((BATCH, SEQ, OUTPUT_DIM), jnp.float32),
        mesh=mesh,
        scratch_types=[
            pltpu.VMEM((chunks_per_w, HALF), jnp.int32),
            pltpu.VMEM((SEQ, OUTPUT_DIM), jnp.float32),
            pltpu.VMEM((SEQ, OUTPUT_DIM), jnp.float32),
            pltpu.SemaphoreType.DMA,
        ],
        compiler_params=pltpu.CompilerParams(use_tc_tiling_on_sc=False),
    )
    def emb_kernel(idx_hbm, table_hbm, pos_hbm, out_hbm, idx_v, pos_v,
                   g0, gsem0):
        wid = lax.axis_index("s") * nc + lax.axis_index("c")
        pltpu.sync_copy(idx_hbm.at[wid], idx_v)
        pltpu.sync_copy(pos_hbm, pos_v)
        base = wid * seqs_per_w

        def body(si, carry):
            gb, gsem = g0, gsem0
            cp0 = pltpu.async_copy(
                table_hbm.at[idx_v.at[2 * si]], gb.at[pl.ds(0, HALF)], gsem)
            cp1 = pltpu.async_copy(
                table_hbm.at[idx_v.at[2 * si + 1]],
                gb.at[pl.ds(HALF, HALF)], gsem)
            cp0.wait()
            cp1.wait()

            def row_body(r, c2):
                for c in range(OUTPUT_DIM // 16):
                    sl = pl.ds(c * 16, 16)
                    gb[r, sl] = gb[r, sl] * SCALE + pos_v[r, sl]
                return c2
            lax.fori_loop(0, SEQ, row_body, 0, unroll=4)

            pltpu.sync_copy(gb, out_hbm.at[base + si])
            return carry

        lax.fori_loop(0, seqs_per_w, body, 0)

    return emb_kernel


def kernel(x, table):
    info = plsc.get_sparse_core_info()
    nc, ns = info.num_cores, info.num_subcores
    nw = nc * ns
    idx = x.reshape(nw, (BATCH * SEQ) // nw // HALF, HALF)
    pos = jnp.asarray(_POS)
    return _build_kernel(nc, ns)(idx, table, pos)
